# Initial kernel scaffold; baseline (speedup 1.0000x reference)
import jax
import jax.numpy as jnp
from jax.experimental import pallas as pl


def kernel(x, tables):
    raise NotImplementedError("write your pallas kernel here")


# speedup vs baseline: 3.7446x; 3.7446x over previous
import functools

import numpy as np
import jax
import jax.numpy as jnp
from jax import lax
from jax.experimental import pallas as pl
from jax.experimental.pallas import tpu as pltpu
from jax.experimental.pallas import tpu_sc as plsc

C = 26
V = 100000
DM = 32
NC, NS = 2, 16
NW = NC * NS
LANES = 16

R = 64
IC = R * C
GS = 128
NG = IC // GS


def _body(nrows, x_hbm, off_hbm, tables_hbm, out_hbm,
          xv, offv, idxv, rows, outv, sem):
    wid = lax.axis_index("s") * NC + lax.axis_index("c")
    rows_per_w = nrows // NW
    nchunks = rows_per_w // R

    pltpu.sync_copy(off_hbm, offv)

    def chunk(g, carry):
        base = pl.multiple_of(wid * rows_per_w + g * R, 8)
        ib = pl.multiple_of(base * C, 8)
        pltpu.sync_copy(x_hbm.at[pl.ds(ib, IC)], xv)

        def addoff(i, c2):
            s = pl.ds(i * LANES, LANES)
            idxv[s] = xv[s] + offv[s]
            return c2
        lax.fori_loop(0, IC // LANES, addoff, 0)

        cps = [
            pltpu.async_copy(
                tables_hbm.at[idxv.at[pl.ds(j * GS, GS)]],
                rows.at[pl.ds(j * GS, GS)],
                sem,
            )
            for j in range(NG)
        ]
        for cp in cps:
            cp.wait()

        def srow(r, c2):
            rb = r * C
            a0 = rows[rb, pl.ds(0, LANES)]
            a1 = rows[rb, pl.ds(LANES, LANES)]
            for c in range(1, C):
                a0 = a0 + rows[rb + c, pl.ds(0, LANES)]
                a1 = a1 + rows[rb + c, pl.ds(LANES, LANES)]
            outv[r, pl.ds(0, LANES)] = a0
            outv[r, pl.ds(LANES, LANES)] = a1
            return c2
        lax.fori_loop(0, R, srow, 0)

        pltpu.sync_copy(outv, out_hbm.at[pl.ds(base, R)])
        return carry

    lax.fori_loop(0, nchunks, chunk, 0)


@functools.partial(jax.jit, static_argnames=())
def kernel(x, tables):
    B, L, c = x.shape
    assert c == C and tables.shape == (C, V, DM)
    N = B * L
    assert N % (NW * R) == 0

    x_flat = x.reshape(N * C)
    tables_flat = tables.reshape(C * V, DM)
    off = jnp.asarray(np.tile(np.arange(C, dtype=np.int32) * V, R))

    mesh = plsc.VectorSubcoreMesh(core_axis_name="c", subcore_axis_name="s")
    call = pl.kernel(
        functools.partial(_body, N),
        out_type=jax.ShapeDtypeStruct((N, DM), jnp.float32),
        mesh=mesh,
        compiler_params=pltpu.CompilerParams(use_tc_tiling_on_sc=False),
        scratch_types=[
            pltpu.VMEM((IC,), jnp.int32),
            pltpu.VMEM((IC,), jnp.int32),
            pltpu.VMEM((IC,), jnp.int32),
            pltpu.VMEM((IC, DM), jnp.float32),
            pltpu.VMEM((R, DM), jnp.float32),
            pltpu.SemaphoreType.DMA,
        ],
    )
    out = call(x_flat, off, tables_flat)
    return out.reshape(B, L, DM)


# speedup vs baseline: 3.8571x; 1.0300x over previous
import functools

import jax
import jax.numpy as jnp
from jax import lax
from jax.experimental import pallas as pl
from jax.experimental.pallas import tpu as pltpu
from jax.experimental.pallas import tpu_sc as plsc

C = 26
V = 100000
DM = 32
L = 20
NC, NS = 2, 16
NW = NC * NS
LANES = 16

NBC = 16
LW = 5
NWAVE = L // LW
RW = LW * NBC
IC = RW * C
GS = 104
NG = IC // GS


def _body(batch, x_hbm, tables_hbm, out_hbm, xv, idxv, rows, outv, sem):
    wid = lax.axis_index("s") * NC + lax.axis_index("c")
    b_per_w = batch // NW
    nchunks = b_per_w // NBC

    def chunk(g, carry):
        b0 = wid * b_per_w + g * NBC
        pltpu.sync_copy(x_hbm.at[:, :, pl.ds(b0, NBC)], xv)

        for w in range(NWAVE):
            def mkidx(t, c2):
                c = t // LW
                dl = t - c * LW
                idxv[pl.ds(t * LANES, LANES)] = xv[c, w * LW + dl, :] + c * V
                return c2
            lax.fori_loop(0, IC // LANES, mkidx, 0)

            cps = [
                pltpu.async_copy(
                    tables_hbm.at[idxv.at[pl.ds(j * GS, GS)]],
                    rows.at[pl.ds(j * GS, GS)],
                    sem,
                )
                for j in range(NG)
            ]
            for cp in cps:
                cp.wait()

            def srow(q, c2):
                dl = q // NBC
                db = q - dl * NBC
                a0 = rows[q, pl.ds(0, LANES)]
                a1 = rows[q, pl.ds(LANES, LANES)]
                for c in range(1, C):
                    a0 = a0 + rows[q + RW * c, pl.ds(0, LANES)]
                    a1 = a1 + rows[q + RW * c, pl.ds(LANES, LANES)]
                outv[db, w * LW + dl, pl.ds(0, LANES)] = a0
                outv[db, w * LW + dl, pl.ds(LANES, LANES)] = a1
                return c2
            lax.fori_loop(0, RW, srow, 0)

        pltpu.sync_copy(outv, out_hbm.at[pl.ds(b0, NBC)])
        return carry

    lax.fori_loop(0, nchunks, chunk, 0)


@jax.jit
def kernel(x, tables):
    B, sl, c = x.shape
    assert c == C and sl == L and tables.shape == (C, V, DM)
    assert B % (NW * NBC) == 0

    xt = jnp.transpose(x, (2, 1, 0))
    tables_flat = tables.reshape(C * V, DM)

    mesh = plsc.VectorSubcoreMesh(core_axis_name="c", subcore_axis_name="s")
    call = pl.kernel(
        functools.partial(_body, B),
        out_type=jax.ShapeDtypeStruct((B, L, DM), jnp.float32),
        mesh=mesh,
        compiler_params=pltpu.CompilerParams(use_tc_tiling_on_sc=False),
        scratch_types=[
            pltpu.VMEM((C, L, NBC), jnp.int32),
            pltpu.VMEM((IC,), jnp.int32),
            pltpu.VMEM((IC, DM), jnp.float32),
            pltpu.VMEM((NBC, L, DM), jnp.float32),
            pltpu.SemaphoreType.DMA,
        ],
    )
    return call(xt, tables_flat)
